# baseline (speedup 1.0000x reference)
import functools

import jax
import jax.numpy as jnp
from jax import lax
from jax.experimental import pallas as pl
from jax.experimental.pallas import tpu as pltpu
from jax.experimental.pallas import tpu_sc as plsc

N_EMB = 8192
DIM = 256
N_TOK = 16384
COMMIT = 0.25

BI = 256
NB = N_TOK // BI

BO = 256
NBO = N_TOK // BO

BI2 = 1024
NB2 = N_TOK // BI2

_NC, _NS = 2, 16
_NW = _NC * _NS
_BPW = N_TOK // _NW
_NCH = 4
_CH = _BPW // _NCH


def _e2_body(e_ref, e2_ref):
    e = e_ref[...]
    e2_ref[...] = jnp.sum(e * e, axis=1).reshape(1, N_EMB)


def _main_body(inp_ref, w_ref, b_ref, e_ref, e2_ref, x_ref, idx_ref):
    xcat = jnp.concatenate([inp_ref[0], inp_ref[1]], axis=1)
    x = lax.dot_general(xcat, w_ref[...],
                        (((1,), (1,)), ((), ()))) + b_ref[...]
    x_ref[...] = x
    xs = jnp.sum(x * x, axis=1, keepdims=True)
    s = lax.dot_general(x, e_ref[...], (((1,), (1,)), ((), ())))
    d = (xs + e2_ref[...]) - 2.0 * s
    dmin = jnp.min(d, axis=1, keepdims=True)
    jio = lax.broadcasted_iota(jnp.int32, (BI, N_EMB), 1)
    idx = jnp.min(jnp.where(d == dmin, jio, N_EMB), axis=1).astype(jnp.int32)
    idx_ref[...] = idx.reshape(1, 1, BI)


def _onehot_body(idx_ref, enc_ref, cnt_ref):
    idx = idx_ref[0, 0, :]
    jio = lax.broadcasted_iota(jnp.int32, (BO, N_EMB), 1)
    enc = (jio == idx[:, None]).astype(jnp.float32)
    enc_ref[...] = enc

    @pl.when(pl.program_id(0) == 0)
    def _init():
        cnt_ref[...] = jnp.zeros_like(cnt_ref)

    cnt_ref[...] += jnp.sum(enc, axis=0).reshape(1, N_EMB)


def _fin_body(x_ref, q_ref, cnt_ref, qst_ref, loss_ref, perp_ref):
    x = x_ref[...]
    q = q_ref[...]
    dlt = q - x
    qst_ref[...] = x + dlt

    @pl.when(pl.program_id(0) == 0)
    def _init():
        loss_ref[0, 0] = 0.0

    loss_ref[0, 0] += jnp.sum(dlt * dlt)

    @pl.when(pl.program_id(0) == pl.num_programs(0) - 1)
    def _done():
        loss_ref[0, 0] = loss_ref[0, 0] * (COMMIT / (N_TOK * DIM))
        p = cnt_ref[...] * (1.0 / N_TOK)
        perp_ref[0, 0] = jnp.exp(-jnp.sum(p * jnp.log(p + 1e-10)))


def _sc_gather_body(e_hbm, idx_hbm, out_hbm, idx0, idx1, rows0, rows1,
                    sg0, sg1):
    wid = lax.axis_index("s") * _NC + lax.axis_index("c")
    base = wid * _BPW
    idxb = (idx0, idx1)
    rows = (rows0, rows1)
    sg = (sg0, sg1)
    gth = [None, None]
    for c in range(_NCH):
        s = c % 2
        if gth[s] is not None:
            gth[s].wait()
            pltpu.sync_copy(rows[s], out_hbm.at[pl.ds(base + (c - 2) * _CH, _CH)])
        pltpu.sync_copy(idx_hbm.at[pl.ds(base + c * _CH, _CH)], idxb[s])
        gth[s] = pltpu.async_copy(e_hbm.at[idxb[s]], rows[s], sg[s])
    for c in range(_NCH - 2, _NCH):
        s = c % 2
        gth[s].wait()
        pltpu.sync_copy(rows[s], out_hbm.at[pl.ds(base + c * _CH, _CH)])


@functools.lru_cache(maxsize=1)
def _sc_gather():
    return pl.kernel(
        _sc_gather_body,
        out_type=jax.ShapeDtypeStruct((N_TOK, DIM), jnp.float32),
        mesh=plsc.VectorSubcoreMesh(
            core_axis_name="c", subcore_axis_name="s",
            num_cores=_NC, num_subcores=_NS),
        scratch_types=[
            pltpu.VMEM((_CH,), jnp.int32),
            pltpu.VMEM((_CH,), jnp.int32),
            pltpu.VMEM((_CH, DIM), jnp.float32),
            pltpu.VMEM((_CH, DIM), jnp.float32),
            pltpu.SemaphoreType.DMA,
            pltpu.SemaphoreType.DMA,
        ],
        name="sc_codebook_gather",
    )


def kernel(inputs, W, b, E):
    b2 = b.reshape(1, DIM)

    e2 = pl.pallas_call(
        _e2_body,
        out_shape=jax.ShapeDtypeStruct((1, N_EMB), jnp.float32),
    )(E)

    x, idx3 = pl.pallas_call(
        _main_body,
        grid=(NB,),
        in_specs=[
            pl.BlockSpec((2, BI, 128), lambda i: (0, i, 0)),
            pl.BlockSpec((DIM, DIM), lambda i: (0, 0)),
            pl.BlockSpec((1, DIM), lambda i: (0, 0)),
            pl.BlockSpec((N_EMB, DIM), lambda i: (0, 0)),
            pl.BlockSpec((1, N_EMB), lambda i: (0, 0)),
        ],
        out_specs=[
            pl.BlockSpec((BI, DIM), lambda i: (i, 0)),
            pl.BlockSpec((1, 1, BI), lambda i: (i, 0, 0)),
        ],
        out_shape=[
            jax.ShapeDtypeStruct((N_TOK, DIM), jnp.float32),
            jax.ShapeDtypeStruct((NB, 1, BI), jnp.int32),
        ],
    )(inputs, W, b2, E, e2)

    idx = idx3.reshape(N_TOK)
    q = _sc_gather()(E, idx)

    idxo = idx3.reshape(NBO, 1, BO)
    enc, cnt = pl.pallas_call(
        _onehot_body,
        grid=(NBO,),
        in_specs=[
            pl.BlockSpec((1, 1, BO), lambda i: (i, 0, 0)),
        ],
        out_specs=[
            pl.BlockSpec((BO, N_EMB), lambda i: (i, 0)),
            pl.BlockSpec((1, N_EMB), lambda i: (0, 0)),
        ],
        out_shape=[
            jax.ShapeDtypeStruct((N_TOK, N_EMB), jnp.float32),
            jax.ShapeDtypeStruct((1, N_EMB), jnp.float32),
        ],
    )(idxo)

    qst, loss, perp = pl.pallas_call(
        _fin_body,
        grid=(NB2,),
        in_specs=[
            pl.BlockSpec((BI2, DIM), lambda i: (i, 0)),
            pl.BlockSpec((BI2, DIM), lambda i: (i, 0)),
            pl.BlockSpec((1, N_EMB), lambda i: (0, 0)),
        ],
        out_specs=[
            pl.BlockSpec((BI2, DIM), lambda i: (i, 0)),
            pl.BlockSpec(memory_space=pltpu.SMEM),
            pl.BlockSpec(memory_space=pltpu.SMEM),
        ],
        out_shape=[
            jax.ShapeDtypeStruct((N_TOK, DIM), jnp.float32),
            jax.ShapeDtypeStruct((1, 1), jnp.float32),
            jax.ShapeDtypeStruct((1, 1), jnp.float32),
        ],
    )(x, q, cnt)

    quantized_out = qst.reshape(2, N_TOK, 128)
    return (loss.reshape(()), quantized_out, perp.reshape(()), enc)


# speedup vs baseline: 1.1194x; 1.1194x over previous
import functools

import jax
import jax.numpy as jnp
from jax import lax
from jax.experimental import pallas as pl
from jax.experimental.pallas import tpu as pltpu
from jax.experimental.pallas import tpu_sc as plsc

N_EMB = 8192
DIM = 256
N_TOK = 16384
COMMIT = 0.25

BI = 256
NB = N_TOK // BI

BO = 256
NBO = N_TOK // BO

BI2 = 1024
NB2 = N_TOK // BI2

_NC, _NS = 2, 16
_NW = _NC * _NS
_BPW = N_TOK // _NW
_NCH = 4
_CH = _BPW // _NCH


def _e2_body(e_ref, e2_ref):
    e = e_ref[...]
    e2_ref[...] = jnp.sum(e * e, axis=1).reshape(1, N_EMB)


def _main_body(inp_ref, w_ref, b_ref, e_ref, e2_ref,
               x_ref, idx_ref, enc_ref, cnt_ref):
    xcat = jnp.concatenate([inp_ref[0], inp_ref[1]], axis=1)
    x = lax.dot_general(xcat, w_ref[...],
                        (((1,), (1,)), ((), ()))) + b_ref[...]
    x_ref[...] = x
    xs = jnp.sum(x * x, axis=1, keepdims=True)
    s = lax.dot_general(x, e_ref[...], (((1,), (1,)), ((), ())))
    d = (xs + e2_ref[...]) - 2.0 * s
    dmin = jnp.min(d, axis=1, keepdims=True)
    jio = lax.broadcasted_iota(jnp.int32, (BI, N_EMB), 1)
    idx = jnp.min(jnp.where(d == dmin, jio, N_EMB), axis=1).astype(jnp.int32)
    idx_ref[...] = idx.reshape(1, 1, BI)
    enc = (jio == idx[:, None]).astype(jnp.float32)
    enc_ref[...] = enc

    @pl.when(pl.program_id(0) == 0)
    def _init():
        cnt_ref[...] = jnp.zeros_like(cnt_ref)

    cnt_ref[...] += jnp.sum(enc, axis=0).reshape(1, N_EMB)


def _fin_body(x_ref, q_ref, cnt_ref, qst_ref, loss_ref, perp_ref):
    x = x_ref[...]
    q = q_ref[...]
    dlt = q - x
    qst_ref[...] = x + dlt

    @pl.when(pl.program_id(0) == 0)
    def _init():
        loss_ref[0, 0] = 0.0

    loss_ref[0, 0] += jnp.sum(dlt * dlt)

    @pl.when(pl.program_id(0) == pl.num_programs(0) - 1)
    def _done():
        loss_ref[0, 0] = loss_ref[0, 0] * (COMMIT / (N_TOK * DIM))
        p = cnt_ref[...] * (1.0 / N_TOK)
        perp_ref[0, 0] = jnp.exp(-jnp.sum(p * jnp.log(p + 1e-10)))


_DEPTH = 3


def _sc_gather_body(e_hbm, idx_hbm, out_hbm,
                    idx0, idx1, idx2, rows0, rows1, rows2,
                    sg0, sg1, sg2, ss0, ss1, ss2):
    wid = lax.axis_index("s") * _NC + lax.axis_index("c")
    base = wid * _BPW
    idxb = (idx0, idx1, idx2)
    rows = (rows0, rows1, rows2)
    sg = (sg0, sg1, sg2)
    ss = (ss0, ss1, ss2)
    gth = [None] * _DEPTH
    sto = [None] * _DEPTH

    def _start(c):
        s = c % _DEPTH
        if sto[s] is not None:
            sto[s].wait()
        pltpu.sync_copy(idx_hbm.at[pl.ds(base + c * _CH, _CH)], idxb[s])
        gth[s] = pltpu.async_copy(e_hbm.at[idxb[s]], rows[s], sg[s])

    for c in range(min(_DEPTH - 1, _NCH)):
        _start(c)
    for c in range(_NCH):
        if c + _DEPTH - 1 < _NCH:
            _start(c + _DEPTH - 1)
        s = c % _DEPTH
        gth[s].wait()
        sto[s] = pltpu.async_copy(
            rows[s], out_hbm.at[pl.ds(base + c * _CH, _CH)], ss[s])
    for c in range(max(0, _NCH - _DEPTH), _NCH):
        sto[c % _DEPTH].wait()


@functools.lru_cache(maxsize=1)
def _sc_gather():
    return pl.kernel(
        _sc_gather_body,
        out_type=jax.ShapeDtypeStruct((N_TOK, DIM), jnp.float32),
        mesh=plsc.VectorSubcoreMesh(
            core_axis_name="c", subcore_axis_name="s",
            num_cores=_NC, num_subcores=_NS),
        scratch_types=[
            pltpu.VMEM((_CH,), jnp.int32),
            pltpu.VMEM((_CH,), jnp.int32),
            pltpu.VMEM((_CH,), jnp.int32),
            pltpu.VMEM((_CH, DIM), jnp.float32),
            pltpu.VMEM((_CH, DIM), jnp.float32),
            pltpu.VMEM((_CH, DIM), jnp.float32),
            pltpu.SemaphoreType.DMA,
            pltpu.SemaphoreType.DMA,
            pltpu.SemaphoreType.DMA,
            pltpu.SemaphoreType.DMA,
            pltpu.SemaphoreType.DMA,
            pltpu.SemaphoreType.DMA,
        ],
        name="sc_codebook_gather",
    )


def kernel(inputs, W, b, E):
    b2 = b.reshape(1, DIM)

    e2 = pl.pallas_call(
        _e2_body,
        out_shape=jax.ShapeDtypeStruct((1, N_EMB), jnp.float32),
    )(E)

    x, idx3, enc, cnt = pl.pallas_call(
        _main_body,
        grid=(NB,),
        in_specs=[
            pl.BlockSpec((2, BI, 128), lambda i: (0, i, 0)),
            pl.BlockSpec((DIM, DIM), lambda i: (0, 0)),
            pl.BlockSpec((1, DIM), lambda i: (0, 0)),
            pl.BlockSpec((N_EMB, DIM), lambda i: (0, 0)),
            pl.BlockSpec((1, N_EMB), lambda i: (0, 0)),
        ],
        out_specs=[
            pl.BlockSpec((BI, DIM), lambda i: (i, 0)),
            pl.BlockSpec((1, 1, BI), lambda i: (i, 0, 0)),
            pl.BlockSpec((BI, N_EMB), lambda i: (i, 0)),
            pl.BlockSpec((1, N_EMB), lambda i: (0, 0)),
        ],
        out_shape=[
            jax.ShapeDtypeStruct((N_TOK, DIM), jnp.float32),
            jax.ShapeDtypeStruct((NB, 1, BI), jnp.int32),
            jax.ShapeDtypeStruct((N_TOK, N_EMB), jnp.float32),
            jax.ShapeDtypeStruct((1, N_EMB), jnp.float32),
        ],
    )(inputs, W, b2, E, e2)

    idx = idx3.reshape(N_TOK)
    q = _sc_gather()(E, idx)

    qst, loss, perp = pl.pallas_call(
        _fin_body,
        grid=(NB2,),
        in_specs=[
            pl.BlockSpec((BI2, DIM), lambda i: (i, 0)),
            pl.BlockSpec((BI2, DIM), lambda i: (i, 0)),
            pl.BlockSpec((1, N_EMB), lambda i: (0, 0)),
        ],
        out_specs=[
            pl.BlockSpec((BI2, DIM), lambda i: (i, 0)),
            pl.BlockSpec(memory_space=pltpu.SMEM),
            pl.BlockSpec(memory_space=pltpu.SMEM),
        ],
        out_shape=[
            jax.ShapeDtypeStruct((N_TOK, DIM), jnp.float32),
            jax.ShapeDtypeStruct((1, 1), jnp.float32),
            jax.ShapeDtypeStruct((1, 1), jnp.float32),
        ],
    )(x, q, cnt)

    quantized_out = qst.reshape(2, N_TOK, 128)
    return (loss.reshape(()), quantized_out, perp.reshape(()), enc)


# speedup vs baseline: 1.1459x; 1.0237x over previous
import functools

import jax
import jax.numpy as jnp
from jax import lax
from jax.experimental import pallas as pl
from jax.experimental.pallas import tpu as pltpu
from jax.experimental.pallas import tpu_sc as plsc

N_EMB = 8192
DIM = 256
N_TOK = 16384
COMMIT = 0.25

BI = 256
NB = N_TOK // BI

BO = 256
NBO = N_TOK // BO

BI2 = 1024
NB2 = N_TOK // BI2

_NC, _NS = 2, 16
_NW = _NC * _NS
_BPW = N_TOK // _NW
_NCH = 4
_CH = _BPW // _NCH


def _main_body(inp_ref, w_ref, b_ref, e_ref,
               x_ref, idx_ref, enc_ref, cnt_ref, e2_ref):
    @pl.when(pl.program_id(0) == 0)
    def _pre():
        e = e_ref[...]
        e2_ref[...] = jnp.sum(e * e, axis=1).reshape(1, N_EMB)

    xcat = jnp.concatenate([inp_ref[0], inp_ref[1]], axis=1)
    x = lax.dot_general(xcat, w_ref[...],
                        (((1,), (1,)), ((), ()))) + b_ref[...]
    x_ref[...] = x
    xs = jnp.sum(x * x, axis=1, keepdims=True)
    s = lax.dot_general(x, e_ref[...], (((1,), (1,)), ((), ())))
    d = (xs + e2_ref[...]) - 2.0 * s
    dmin = jnp.min(d, axis=1, keepdims=True)
    jio = lax.broadcasted_iota(jnp.int32, (BI, N_EMB), 1)
    idx = jnp.min(jnp.where(d == dmin, jio, N_EMB), axis=1).astype(jnp.int32)
    idx_ref[...] = idx.reshape(1, 1, BI)
    enc = (jio == idx[:, None]).astype(jnp.float32)
    enc_ref[...] = enc

    @pl.when(pl.program_id(0) == 0)
    def _init():
        cnt_ref[...] = jnp.zeros_like(cnt_ref)

    cnt_ref[...] += jnp.sum(enc, axis=0).reshape(1, N_EMB)


def _fin_body(x_ref, q_ref, cnt_ref, loss_ref, perp_ref):
    x = x_ref[...]
    q = q_ref[...]
    dlt = q - x

    @pl.when(pl.program_id(0) == 0)
    def _init():
        loss_ref[0, 0] = 0.0

    loss_ref[0, 0] += jnp.sum(dlt * dlt)

    @pl.when(pl.program_id(0) == pl.num_programs(0) - 1)
    def _done():
        loss_ref[0, 0] = loss_ref[0, 0] * (COMMIT / (N_TOK * DIM))
        p = cnt_ref[...] * (1.0 / N_TOK)
        perp_ref[0, 0] = jnp.exp(-jnp.sum(p * jnp.log(p + 1e-10)))


_DEPTH = 3


def _sc_gather_body(e_hbm, idx_hbm, out_hbm,
                    idx0, idx1, idx2, rows0, rows1, rows2,
                    sg0, sg1, sg2, ss0, ss1, ss2):
    wid = lax.axis_index("s") * _NC + lax.axis_index("c")
    base = wid * _BPW
    idxb = (idx0, idx1, idx2)
    rows = (rows0, rows1, rows2)
    sg = (sg0, sg1, sg2)
    ss = (ss0, ss1, ss2)
    gth = [None] * _DEPTH
    sto = [None] * _DEPTH

    def _start(c):
        s = c % _DEPTH
        if sto[s] is not None:
            sto[s].wait()
        pltpu.sync_copy(idx_hbm.at[pl.ds(base + c * _CH, _CH)], idxb[s])
        gth[s] = pltpu.async_copy(e_hbm.at[idxb[s]], rows[s], sg[s])

    for c in range(min(_DEPTH - 1, _NCH)):
        _start(c)
    for c in range(_NCH):
        if c + _DEPTH - 1 < _NCH:
            _start(c + _DEPTH - 1)
        s = c % _DEPTH
        gth[s].wait()
        sto[s] = pltpu.async_copy(
            rows[s], out_hbm.at[pl.ds(base + c * _CH, _CH)], ss[s])
    for c in range(max(0, _NCH - _DEPTH), _NCH):
        sto[c % _DEPTH].wait()


@functools.lru_cache(maxsize=1)
def _sc_gather():
    return pl.kernel(
        _sc_gather_body,
        out_type=jax.ShapeDtypeStruct((N_TOK, DIM), jnp.float32),
        mesh=plsc.VectorSubcoreMesh(
            core_axis_name="c", subcore_axis_name="s",
            num_cores=_NC, num_subcores=_NS),
        scratch_types=[
            pltpu.VMEM((_CH,), jnp.int32),
            pltpu.VMEM((_CH,), jnp.int32),
            pltpu.VMEM((_CH,), jnp.int32),
            pltpu.VMEM((_CH, DIM), jnp.float32),
            pltpu.VMEM((_CH, DIM), jnp.float32),
            pltpu.VMEM((_CH, DIM), jnp.float32),
            pltpu.SemaphoreType.DMA,
            pltpu.SemaphoreType.DMA,
            pltpu.SemaphoreType.DMA,
            pltpu.SemaphoreType.DMA,
            pltpu.SemaphoreType.DMA,
            pltpu.SemaphoreType.DMA,
        ],
        name="sc_codebook_gather",
    )


def kernel(inputs, W, b, E):
    b2 = b.reshape(1, DIM)

    x, idx3, enc, cnt = pl.pallas_call(
        _main_body,
        grid=(NB,),
        in_specs=[
            pl.BlockSpec((2, BI, 128), lambda i: (0, i, 0)),
            pl.BlockSpec((DIM, DIM), lambda i: (0, 0)),
            pl.BlockSpec((1, DIM), lambda i: (0, 0)),
            pl.BlockSpec((N_EMB, DIM), lambda i: (0, 0)),
        ],
        out_specs=[
            pl.BlockSpec((BI, DIM), lambda i: (i, 0)),
            pl.BlockSpec((1, 1, BI), lambda i: (i, 0, 0)),
            pl.BlockSpec((BI, N_EMB), lambda i: (i, 0)),
            pl.BlockSpec((1, N_EMB), lambda i: (0, 0)),
        ],
        out_shape=[
            jax.ShapeDtypeStruct((N_TOK, DIM), jnp.float32),
            jax.ShapeDtypeStruct((NB, 1, BI), jnp.int32),
            jax.ShapeDtypeStruct((N_TOK, N_EMB), jnp.float32),
            jax.ShapeDtypeStruct((1, N_EMB), jnp.float32),
        ],
        scratch_shapes=[pltpu.VMEM((1, N_EMB), jnp.float32)],
    )(inputs, W, b2, E)

    idx = idx3.reshape(N_TOK)
    q = _sc_gather()(E, idx)

    loss, perp = pl.pallas_call(
        _fin_body,
        grid=(NB2,),
        in_specs=[
            pl.BlockSpec((BI2, DIM), lambda i: (i, 0)),
            pl.BlockSpec((BI2, DIM), lambda i: (i, 0)),
            pl.BlockSpec((1, N_EMB), lambda i: (0, 0)),
        ],
        out_specs=[
            pl.BlockSpec(memory_space=pltpu.SMEM),
            pl.BlockSpec(memory_space=pltpu.SMEM),
        ],
        out_shape=[
            jax.ShapeDtypeStruct((1, 1), jnp.float32),
            jax.ShapeDtypeStruct((1, 1), jnp.float32),
        ],
    )(x, q, cnt)

    quantized_out = q.reshape(2, N_TOK, 128)
    return (loss.reshape(()), quantized_out, perp.reshape(()), enc)


# speedup vs baseline: 1.2182x; 1.0631x over previous
import functools

import jax
import jax.numpy as jnp
from jax import lax
from jax.experimental import pallas as pl
from jax.experimental.pallas import tpu as pltpu
from jax.experimental.pallas import tpu_sc as plsc

N_EMB = 8192
DIM = 256
N_TOK = 16384
COMMIT = 0.25

BI = 512
NB = N_TOK // BI

BO = 256
NBO = N_TOK // BO

BI2 = 1024
NB2 = N_TOK // BI2

_NC, _NS = 2, 16
_NW = _NC * _NS
_BPW = N_TOK // _NW
_NCH = 4
_CH = _BPW // _NCH


def _e2_body(e_ref, e2_ref):
    e = e_ref[...]
    e2_ref[...] = jnp.sum(e * e, axis=1).reshape(1, N_EMB)


def _main_body(inp_ref, w_ref, b_ref, e_ref, e2_ref,
               x_ref, idx_ref, enc_ref, cnt_ref):
    xcat = jnp.concatenate([inp_ref[0], inp_ref[1]], axis=1)
    x = lax.dot_general(xcat, w_ref[...],
                        (((1,), (1,)), ((), ()))) + b_ref[...]
    x_ref[...] = x
    xs = jnp.sum(x * x, axis=1, keepdims=True)
    s = lax.dot_general(x, e_ref[...], (((1,), (1,)), ((), ())))
    d = (xs + e2_ref[...]) - 2.0 * s
    dmin = jnp.min(d, axis=1, keepdims=True)
    jio = lax.broadcasted_iota(jnp.int32, (BI, N_EMB), 1)
    idx = jnp.min(jnp.where(d == dmin, jio, N_EMB), axis=1).astype(jnp.int32)
    idx_ref[...] = idx.reshape(1, 1, BI)
    enc = (jio == idx[:, None]).astype(jnp.float32)
    enc_ref[...] = enc

    @pl.when(pl.program_id(0) == 0)
    def _init():
        cnt_ref[...] = jnp.zeros_like(cnt_ref)

    cnt_ref[...] += jnp.sum(enc, axis=0).reshape(1, N_EMB)


def _fin_body(x_ref, q_ref, cnt_ref, loss_ref, perp_ref):
    x = x_ref[...]
    q = q_ref[...]
    dlt = q - x

    @pl.when(pl.program_id(0) == 0)
    def _init():
        loss_ref[0, 0] = 0.0

    loss_ref[0, 0] += jnp.sum(dlt * dlt)

    @pl.when(pl.program_id(0) == pl.num_programs(0) - 1)
    def _done():
        loss_ref[0, 0] = loss_ref[0, 0] * (COMMIT / (N_TOK * DIM))
        p = cnt_ref[...] * (1.0 / N_TOK)
        perp_ref[0, 0] = jnp.exp(-jnp.sum(p * jnp.log(p + 1e-10)))


_DEPTH = 3


def _sc_gather_body(e_hbm, idx_hbm, out_hbm,
                    idx0, idx1, idx2, rows0, rows1, rows2,
                    sg0, sg1, sg2, ss0, ss1, ss2):
    wid = lax.axis_index("s") * _NC + lax.axis_index("c")
    base = wid * _BPW
    idxb = (idx0, idx1, idx2)
    rows = (rows0, rows1, rows2)
    sg = (sg0, sg1, sg2)
    ss = (ss0, ss1, ss2)
    gth = [None] * _DEPTH
    sto = [None] * _DEPTH

    def _start(c):
        s = c % _DEPTH
        if sto[s] is not None:
            sto[s].wait()
        pltpu.sync_copy(idx_hbm.at[pl.ds(base + c * _CH, _CH)], idxb[s])
        gth[s] = pltpu.async_copy(e_hbm.at[idxb[s]], rows[s], sg[s])

    for c in range(min(_DEPTH - 1, _NCH)):
        _start(c)
    for c in range(_NCH):
        if c + _DEPTH - 1 < _NCH:
            _start(c + _DEPTH - 1)
        s = c % _DEPTH
        gth[s].wait()
        sto[s] = pltpu.async_copy(
            rows[s], out_hbm.at[pl.ds(base + c * _CH, _CH)], ss[s])
    for c in range(max(0, _NCH - _DEPTH), _NCH):
        sto[c % _DEPTH].wait()


@functools.lru_cache(maxsize=1)
def _sc_gather():
    return pl.kernel(
        _sc_gather_body,
        out_type=jax.ShapeDtypeStruct((N_TOK, DIM), jnp.float32),
        mesh=plsc.VectorSubcoreMesh(
            core_axis_name="c", subcore_axis_name="s",
            num_cores=_NC, num_subcores=_NS),
        scratch_types=[
            pltpu.VMEM((_CH,), jnp.int32),
            pltpu.VMEM((_CH,), jnp.int32),
            pltpu.VMEM((_CH,), jnp.int32),
            pltpu.VMEM((_CH, DIM), jnp.float32),
            pltpu.VMEM((_CH, DIM), jnp.float32),
            pltpu.VMEM((_CH, DIM), jnp.float32),
            pltpu.SemaphoreType.DMA,
            pltpu.SemaphoreType.DMA,
            pltpu.SemaphoreType.DMA,
            pltpu.SemaphoreType.DMA,
            pltpu.SemaphoreType.DMA,
            pltpu.SemaphoreType.DMA,
        ],
        name="sc_codebook_gather",
    )


def kernel(inputs, W, b, E):
    b2 = b.reshape(1, DIM)

    e2 = pl.pallas_call(
        _e2_body,
        out_shape=jax.ShapeDtypeStruct((1, N_EMB), jnp.float32),
    )(E)

    x, idx3, enc, cnt = pl.pallas_call(
        _main_body,
        grid=(NB,),
        in_specs=[
            pl.BlockSpec((2, BI, 128), lambda i: (0, i, 0)),
            pl.BlockSpec((DIM, DIM), lambda i: (0, 0)),
            pl.BlockSpec((1, DIM), lambda i: (0, 0)),
            pl.BlockSpec((N_EMB, DIM), lambda i: (0, 0)),
            pl.BlockSpec((1, N_EMB), lambda i: (0, 0)),
        ],
        out_specs=[
            pl.BlockSpec((BI, DIM), lambda i: (i, 0)),
            pl.BlockSpec((1, 1, BI), lambda i: (i, 0, 0)),
            pl.BlockSpec((BI, N_EMB), lambda i: (i, 0)),
            pl.BlockSpec((1, N_EMB), lambda i: (0, 0)),
        ],
        out_shape=[
            jax.ShapeDtypeStruct((N_TOK, DIM), jnp.float32),
            jax.ShapeDtypeStruct((NB, 1, BI), jnp.int32),
            jax.ShapeDtypeStruct((N_TOK, N_EMB), jnp.float32),
            jax.ShapeDtypeStruct((1, N_EMB), jnp.float32),
        ],
    )(inputs, W, b2, E, e2)

    idx = idx3.reshape(N_TOK)
    q = _sc_gather()(E, idx)

    loss, perp = pl.pallas_call(
        _fin_body,
        grid=(NB2,),
        in_specs=[
            pl.BlockSpec((BI2, DIM), lambda i: (i, 0)),
            pl.BlockSpec((BI2, DIM), lambda i: (i, 0)),
            pl.BlockSpec((1, N_EMB), lambda i: (0, 0)),
        ],
        out_specs=[
            pl.BlockSpec(memory_space=pltpu.SMEM),
            pl.BlockSpec(memory_space=pltpu.SMEM),
        ],
        out_shape=[
            jax.ShapeDtypeStruct((1, 1), jnp.float32),
            jax.ShapeDtypeStruct((1, 1), jnp.float32),
        ],
    )(x, q, cnt)

    quantized_out = q.reshape(2, N_TOK, 128)
    return (loss.reshape(()), quantized_out, perp.reshape(()), enc)
